# Initial kernel scaffold; baseline (speedup 1.0000x reference)
import jax
import jax.numpy as jnp
from jax.experimental import pallas as pl


def kernel(x, pop_ids, W, b):
    raise NotImplementedError("write your pallas kernel here")


# speedup vs baseline: 1.8007x; 1.8007x over previous
import jax
import jax.numpy as jnp
from jax.experimental import pallas as pl

N_TOKENS = 8192
HIDDEN = 4096
NPOP = 8
BLK = 512
NBLK = N_TOKENS // BLK


def _values_kernel(ids_ref, x_ref, w_ref, b_ref, out_ref):
    xb = x_ref[...]
    w = w_ref[...]
    ids = ids_ref[0, 0, :]
    logits = jax.lax.dot_general(
        xb, w, (((1,), (1,)), ((), ())),
        preferred_element_type=jnp.float32)
    onehot = (ids[:, None] == jax.lax.iota(jnp.int32, NPOP)[None, :]
              ).astype(jnp.float32)
    vals = jnp.sum((logits + b_ref[...][None, :]) * onehot,
                   axis=1, keepdims=True)
    out_ref[...] = vals


def kernel(x, pop_ids, W, b):
    ids3 = pop_ids.reshape(NBLK, 1, BLK)
    values = pl.pallas_call(
        _values_kernel,
        grid=(NBLK,),
        in_specs=[
            pl.BlockSpec((1, 1, BLK), lambda i: (i, 0, 0)),
            pl.BlockSpec((BLK, HIDDEN), lambda i: (i, 0)),
            pl.BlockSpec((NPOP, HIDDEN), lambda i: (0, 0)),
            pl.BlockSpec((NPOP,), lambda i: (0,)),
        ],
        out_specs=pl.BlockSpec((BLK, 1), lambda i: (i, 0)),
        out_shape=jax.ShapeDtypeStruct((N_TOKENS, 1), jnp.float32),
    )(ids3, x, W, b)
    return (x, values)


# speedup vs baseline: 2.6320x; 1.4617x over previous
import jax
import jax.numpy as jnp
from jax.experimental import pallas as pl

N_TOKENS = 8192
HIDDEN = 4096
NPOP = 8
BLK = 512
NBLK = N_TOKENS // BLK


def _values_kernel(ids_ref, x_ref, w_ref, b_ref, hid_ref, out_ref):
    xb = x_ref[...]
    hid_ref[...] = xb
    w = w_ref[...]
    ids = ids_ref[0, 0, :]
    logits = jax.lax.dot_general(
        xb, w, (((1,), (1,)), ((), ())),
        preferred_element_type=jnp.float32)
    onehot = (ids[:, None] == jax.lax.iota(jnp.int32, NPOP)[None, :]
              ).astype(jnp.float32)
    vals = jnp.sum((logits + b_ref[...][None, :]) * onehot,
                   axis=1, keepdims=True)
    out_ref[...] = vals


def kernel(x, pop_ids, W, b):
    ids3 = pop_ids.reshape(NBLK, 1, BLK)
    hidden, values = pl.pallas_call(
        _values_kernel,
        grid=(NBLK,),
        in_specs=[
            pl.BlockSpec((1, 1, BLK), lambda i: (i, 0, 0)),
            pl.BlockSpec((BLK, HIDDEN), lambda i: (i, 0)),
            pl.BlockSpec((NPOP, HIDDEN), lambda i: (0, 0)),
            pl.BlockSpec((NPOP,), lambda i: (0,)),
        ],
        out_specs=[
            pl.BlockSpec((BLK, HIDDEN), lambda i: (i, 0)),
            pl.BlockSpec((BLK, 1), lambda i: (i, 0)),
        ],
        out_shape=[
            jax.ShapeDtypeStruct((N_TOKENS, HIDDEN), jnp.float32),
            jax.ShapeDtypeStruct((N_TOKENS, 1), jnp.float32),
        ],
    )(ids3, x, W, b)
    return (hidden, values)
